# Initial kernel scaffold; baseline (speedup 1.0000x reference)
import jax
import jax.numpy as jnp
from jax.experimental import pallas as pl


def kernel(x, edge_index, batch, W1, b1, W2, b2, Wc, bc):
    raise NotImplementedError("write your pallas kernel here")


# speedup vs baseline: 15.5030x; 15.5030x over previous
import functools

import jax
import jax.numpy as jnp
from jax import lax
from jax.experimental import pallas as pl
from jax.experimental.pallas import tpu as pltpu
from jax.experimental.pallas import tpu_sc as plsc

N_NODES = 10000
D = 128
NG = 64
NC = 2
NS = 16
NW = NC * NS
CH = 128

R = 400
GRID = N_NODES // R
N_PAD = 10240
DEG_PAD = 10240


def _sc_degree(dst):
    E = dst.shape[0]
    ept = E // NW
    nch = ept // CH
    tail = ept - nch * CH
    mesh = plsc.VectorSubcoreMesh(core_axis_name="c", subcore_axis_name="s")

    @functools.partial(
        pl.kernel,
        out_type=jax.ShapeDtypeStruct((NC, DEG_PAD), jnp.float32),
        mesh=mesh,
        scratch_types=[
            pltpu.VMEM((640,), jnp.float32),
            pltpu.VMEM((CH,), jnp.float32),
            pltpu.VMEM((CH,), jnp.int32),
            pltpu.VMEM((16,), jnp.int32),
            pltpu.VMEM_SHARED((DEG_PAD,), jnp.float32),
        ],
    )
    def k(dst_ref, out_ref, zbuf, ones, didx, didx_t, acc):
        c = lax.axis_index("c")
        s = lax.axis_index("s")
        wid = c * NS + s

        def zfill(i, carry):
            zbuf[pl.ds(i * 16, 16)] = jnp.zeros((16,), jnp.float32)
            return carry

        lax.fori_loop(0, 40, zfill, 0)
        for i in range(CH // 16):
            ones[pl.ds(i * 16, 16)] = jnp.ones((16,), jnp.float32)
        pltpu.sync_copy(zbuf, acc.at[pl.ds(s * 640, 640)])
        plsc.subcore_barrier()

        def body(j, carry):
            base = pl.multiple_of(wid * ept + j * CH, 8)
            pltpu.sync_copy(dst_ref.at[pl.ds(base, CH)], didx)
            pltpu.sync_copy(ones, acc.at[didx], add=True)
            return carry

        lax.fori_loop(0, nch, body, 0)
        if tail:
            base = pl.multiple_of(wid * ept + nch * CH, 8)
            pltpu.sync_copy(dst_ref.at[pl.ds(base, tail)], didx_t)
            pltpu.sync_copy(ones.at[pl.ds(0, tail)], acc.at[didx_t], add=True)
        plsc.subcore_barrier()
        pltpu.sync_copy(acc.at[pl.ds(s * 640, 640)],
                        out_ref.at[c, pl.ds(s * 640, 640)])

    return k(dst)


def _sc_aggregate(g, src, dst):
    E = src.shape[0]
    ept = E // NW
    nch = ept // CH
    tail = ept - nch * CH
    rpt = N_PAD // NS
    mesh = plsc.VectorSubcoreMesh(core_axis_name="c", subcore_axis_name="s")

    @functools.partial(
        pl.kernel,
        out_type=jax.ShapeDtypeStruct((NC, N_PAD, D), jnp.float32),
        mesh=mesh,
        scratch_types=[
            pltpu.VMEM((CH, D), jnp.float32),
            pltpu.VMEM((CH,), jnp.int32),
            pltpu.VMEM((CH,), jnp.int32),
            pltpu.VMEM((16,), jnp.int32),
            pltpu.VMEM((16,), jnp.int32),
            pltpu.VMEM_SHARED((N_PAD, D), jnp.float32),
            pltpu.SemaphoreType.DMA,
        ],
    )
    def k(g_ref, src_ref, dst_ref, out_ref, rows, sidx, didx, sidx_t, didx_t,
          acc, sem):
        c = lax.axis_index("c")
        s = lax.axis_index("s")
        wid = c * NS + s

        def zrow(i, carry):
            for k16 in range(D // 16):
                rows[i, pl.ds(k16 * 16, 16)] = jnp.zeros((16,), jnp.float32)
            return carry

        lax.fori_loop(0, CH, zrow, 0)
        rbase = s * rpt
        nfull = rpt // CH
        for t in range(nfull):
            pltpu.sync_copy(rows, acc.at[pl.ds(rbase + t * CH, CH)])
        rrem = rpt - nfull * CH
        if rrem:
            pltpu.sync_copy(rows.at[pl.ds(0, rrem)],
                            acc.at[pl.ds(rbase + nfull * CH, rrem)])
        plsc.subcore_barrier()

        def body(j, carry):
            base = pl.multiple_of(wid * ept + j * CH, 8)
            pltpu.sync_copy(src_ref.at[pl.ds(base, CH)], sidx)
            pltpu.sync_copy(dst_ref.at[pl.ds(base, CH)], didx)
            pltpu.async_copy(g_ref.at[sidx], rows, sem).wait()
            pltpu.sync_copy(rows, acc.at[didx], add=True)
            return carry

        lax.fori_loop(0, nch, body, 0)
        if tail:
            base = pl.multiple_of(wid * ept + nch * CH, 8)
            pltpu.sync_copy(src_ref.at[pl.ds(base, tail)], sidx_t)
            pltpu.sync_copy(dst_ref.at[pl.ds(base, tail)], didx_t)
            pltpu.async_copy(g_ref.at[sidx_t], rows.at[pl.ds(0, tail)],
                             sem).wait()
            pltpu.sync_copy(rows.at[pl.ds(0, tail)], acc.at[didx_t], add=True)
        plsc.subcore_barrier()
        pltpu.sync_copy(acc.at[pl.ds(rbase, rpt)],
                        out_ref.at[c, pl.ds(rbase, rpt)])

    return k(g, src, dst)


def _dinv_block(d0, d1):
    deg = d0 + d1
    return lax.rsqrt(jnp.maximum(deg, 1.0))


def _tc1_body(x_ref, w_ref, d0_ref, d1_ref, g_ref):
    dinv = _dinv_block(d0_ref[...], d1_ref[...])
    h = jnp.dot(x_ref[...], w_ref[...], preferred_element_type=jnp.float32)
    g_ref[...] = h * dinv


def _tc2_body(a0_ref, a1_ref, g1_ref, d0_ref, d1_ref, w_ref, b_ref, g2_ref):
    dinv = _dinv_block(d0_ref[...], d1_ref[...])
    x2 = jnp.maximum(
        dinv * (a0_ref[...] + a1_ref[...] + g1_ref[...]) + b_ref[...], 0.0)
    h = jnp.dot(x2, w_ref[...], preferred_element_type=jnp.float32)
    g2_ref[...] = h * dinv


def _tc3_body(a0_ref, a1_ref, g2_ref, d0_ref, d1_ref, b_ref, batch_ref,
              wc_ref, bc_ref, out_ref, sums, cnts):
    i = pl.program_id(0)
    dinv = _dinv_block(d0_ref[...], d1_ref[...])
    h3 = jnp.maximum(
        dinv * (a0_ref[...] + a1_ref[...] + g2_ref[...]) + b_ref[...], 0.0)
    bb = batch_ref[0, 0, :]
    onehot = jnp.equal(
        jnp.reshape(bb, (R, 1)),
        lax.broadcasted_iota(jnp.int32, (R, NG), 1)).astype(jnp.float32)
    ps = lax.dot_general(onehot, h3, (((0,), (0,)), ((), ())),
                         preferred_element_type=jnp.float32)
    pc = lax.dot_general(onehot, jnp.ones((R, D), jnp.float32),
                         (((0,), (0,)), ((), ())),
                         preferred_element_type=jnp.float32)

    @pl.when(i == 0)
    def _():
        sums[...] = ps
        cnts[...] = pc

    @pl.when(i > 0)
    def _():
        sums[...] += ps
        cnts[...] += pc

    @pl.when(i == GRID - 1)
    def _():
        pooled = sums[...] / jnp.maximum(cnts[...], 1.0)
        logits = jnp.dot(pooled, wc_ref[...],
                         preferred_element_type=jnp.float32) + bc_ref[...]
        m = jnp.max(logits, axis=1, keepdims=True)
        sh = logits - m
        lse = jnp.log(jnp.sum(jnp.exp(sh), axis=1, keepdims=True))
        out_ref[...] = sh - lse


def kernel(x, edge_index, batch, W1, b1, W2, b2, Wc, bc):
    src = edge_index[0]
    dst = edge_index[1]

    deg = _sc_degree(dst)
    d0 = jnp.reshape(deg[0], (DEG_PAD, 1))
    d1 = jnp.reshape(deg[1], (DEG_PAD, 1))

    row_spec = pl.BlockSpec((R, D), lambda i: (i, 0))
    dspec = pl.BlockSpec((R, 1), lambda i: (i, 0))
    wspec = pl.BlockSpec((D, D), lambda i: (0, 0))
    bspec = pl.BlockSpec((1, D), lambda i: (0, 0))

    g1 = pl.pallas_call(
        _tc1_body,
        grid=(GRID,),
        in_specs=[row_spec, wspec, dspec, dspec],
        out_specs=row_spec,
        out_shape=jax.ShapeDtypeStruct((N_NODES, D), jnp.float32),
    )(x, W1, d0, d1)

    a1 = _sc_aggregate(g1, src, dst)

    g2 = pl.pallas_call(
        _tc2_body,
        grid=(GRID,),
        in_specs=[row_spec, row_spec, row_spec, dspec, dspec, wspec, bspec],
        out_specs=row_spec,
        out_shape=jax.ShapeDtypeStruct((N_NODES, D), jnp.float32),
    )(a1[0], a1[1], g1, d0, d1, W2, jnp.reshape(b1, (1, D)))

    a2 = _sc_aggregate(g2, src, dst)

    batch3 = jnp.reshape(batch, (GRID, 1, R))
    wc_pad = jnp.zeros((D, D), jnp.float32).at[:, :Wc.shape[1]].set(Wc)
    bc_pad = jnp.full((1, D), -1e30, jnp.float32).at[0, :bc.shape[0]].set(bc)

    logits_pad = pl.pallas_call(
        _tc3_body,
        grid=(GRID,),
        in_specs=[row_spec, row_spec, row_spec, dspec, dspec, bspec,
                  pl.BlockSpec((1, 1, R), lambda i: (i, 0, 0)),
                  wspec, bspec],
        out_specs=pl.BlockSpec((NG, D), lambda i: (0, 0)),
        out_shape=jax.ShapeDtypeStruct((NG, D), jnp.float32),
        scratch_shapes=[pltpu.VMEM((NG, D), jnp.float32),
                        pltpu.VMEM((NG, D), jnp.float32)],
    )(a2[0], a2[1], g2, d0, d1, jnp.reshape(b2, (1, D)), batch3, wc_pad,
      bc_pad)

    return logits_pad[:, :bc.shape[0]]


# speedup vs baseline: 24.4872x; 1.5795x over previous
import functools

import jax
import jax.numpy as jnp
from jax import lax
from jax.experimental import pallas as pl
from jax.experimental.pallas import tpu as pltpu
from jax.experimental.pallas import tpu_sc as plsc

N_NODES = 10000
D = 128
NG = 64
NC = 2
NS = 16
NW = NC * NS
CH = 128

R = 400
GRID = N_NODES // R
N_PAD = 10240
DEG_PAD = 10240


def _sc_degree(dst):
    E = dst.shape[0]
    ept = E // NW
    nch = ept // CH
    tail = ept - nch * CH
    mesh = plsc.VectorSubcoreMesh(core_axis_name="c", subcore_axis_name="s")

    @functools.partial(
        pl.kernel,
        out_type=jax.ShapeDtypeStruct((NC, DEG_PAD), jnp.float32),
        mesh=mesh,
        scratch_types=[
            pltpu.VMEM((640,), jnp.float32),
            pltpu.VMEM((CH,), jnp.float32),
            pltpu.VMEM((CH,), jnp.int32),
            pltpu.VMEM((16,), jnp.int32),
            pltpu.VMEM_SHARED((DEG_PAD,), jnp.float32),
        ],
    )
    def k(dst_ref, out_ref, zbuf, ones, didx, didx_t, acc):
        c = lax.axis_index("c")
        s = lax.axis_index("s")
        wid = c * NS + s

        def zfill(i, carry):
            zbuf[pl.ds(i * 16, 16)] = jnp.zeros((16,), jnp.float32)
            return carry

        lax.fori_loop(0, 40, zfill, 0)
        for i in range(CH // 16):
            ones[pl.ds(i * 16, 16)] = jnp.ones((16,), jnp.float32)
        pltpu.sync_copy(zbuf, acc.at[pl.ds(s * 640, 640)])
        plsc.subcore_barrier()

        def body(j, carry):
            base = pl.multiple_of(wid * ept + j * CH, 8)
            pltpu.sync_copy(dst_ref.at[pl.ds(base, CH)], didx)
            pltpu.sync_copy(ones, acc.at[didx], add=True)
            return carry

        lax.fori_loop(0, nch, body, 0)
        if tail:
            base = pl.multiple_of(wid * ept + nch * CH, 8)
            pltpu.sync_copy(dst_ref.at[pl.ds(base, tail)], didx_t)
            pltpu.sync_copy(ones.at[pl.ds(0, tail)], acc.at[didx_t], add=True)
        plsc.subcore_barrier()
        pltpu.sync_copy(acc.at[pl.ds(s * 640, 640)],
                        out_ref.at[c, pl.ds(s * 640, 640)])

    return k(dst)


def _sc_aggregate(g, src, dst):
    E = src.shape[0]
    ACH = 112
    ept = E // NW
    nch = ept // ACH
    tail = ept - nch * ACH
    rpt = N_PAD // NS
    mesh = plsc.VectorSubcoreMesh(core_axis_name="c", subcore_axis_name="s")

    SL = 3

    @functools.partial(
        pl.kernel,
        out_type=jax.ShapeDtypeStruct((NC, N_PAD, D), jnp.float32),
        mesh=mesh,
        scratch_types=[
            pltpu.VMEM((SL, ACH, D), jnp.float32),
            pltpu.VMEM((SL, ACH), jnp.int32),
            pltpu.VMEM((SL, ACH), jnp.int32),
            pltpu.VMEM((tail,), jnp.int32),
            pltpu.VMEM((tail,), jnp.int32),
            pltpu.VMEM_SHARED((N_PAD, D), jnp.float32),
            pltpu.SemaphoreType.DMA((SL,)),
            pltpu.SemaphoreType.DMA((SL,)),
            pltpu.SemaphoreType.DMA,
        ],
    )
    def k(g_ref, src_ref, dst_ref, out_ref, rows, sidx, didx, sidx_t, didx_t,
          acc, gsem, ssem, tsem):
        c = lax.axis_index("c")
        s = lax.axis_index("s")
        wid = c * NS + s

        def zrow(i, carry):
            for k16 in range(D // 16):
                rows[0, i, pl.ds(k16 * 16, 16)] = jnp.zeros((16,), jnp.float32)
            return carry

        lax.fori_loop(0, ACH, zrow, 0)
        rbase = s * rpt
        nfull = rpt // ACH
        for t in range(nfull):
            pltpu.sync_copy(rows.at[0], acc.at[pl.ds(rbase + t * ACH, ACH)])
        rrem = rpt - nfull * ACH
        if rrem:
            pltpu.sync_copy(rows.at[0, pl.ds(0, rrem)],
                            acc.at[pl.ds(rbase + nfull * ACH, rrem)])
        plsc.subcore_barrier()

        ebase = wid * ept
        for p in range(min(2, nch)):
            base = pl.multiple_of(ebase + p * ACH, 8)
            pltpu.sync_copy(src_ref.at[pl.ds(base, ACH)], sidx.at[p])
            pltpu.sync_copy(dst_ref.at[pl.ds(base, ACH)], didx.at[p])
            pltpu.async_copy(g_ref.at[sidx.at[p]], rows.at[p], gsem.at[p])

        def body(j, carry):
            b = j % SL
            pltpu.make_async_copy(g_ref.at[sidx.at[b]], rows.at[b],
                                  gsem.at[b]).wait()
            pltpu.async_copy(rows.at[b], acc.at[didx.at[b]], ssem.at[b],
                             add=True)

            @pl.when(j + 2 < nch)
            def _():
                bn = (j + 2) % SL

                @pl.when(j >= 1)
                def _():
                    pltpu.make_async_copy(rows.at[bn], acc.at[didx.at[bn]],
                                          ssem.at[bn]).wait()

                base = pl.multiple_of(ebase + (j + 2) * ACH, 8)
                pltpu.sync_copy(src_ref.at[pl.ds(base, ACH)], sidx.at[bn])
                pltpu.sync_copy(dst_ref.at[pl.ds(base, ACH)], didx.at[bn])
                pltpu.async_copy(g_ref.at[sidx.at[bn]], rows.at[bn],
                                 gsem.at[bn])

            return carry

        lax.fori_loop(0, nch, body, 0)
        for dj in range(max(nch - 3, 0), nch):
            b = dj % SL
            pltpu.make_async_copy(rows.at[b], acc.at[didx.at[b]],
                                  ssem.at[b]).wait()
        if tail:
            base = pl.multiple_of(ebase + nch * ACH, 8)
            pltpu.sync_copy(src_ref.at[pl.ds(base, tail)], sidx_t)
            pltpu.sync_copy(dst_ref.at[pl.ds(base, tail)], didx_t)
            pltpu.async_copy(g_ref.at[sidx_t], rows.at[0, pl.ds(0, tail)],
                             tsem).wait()
            pltpu.sync_copy(rows.at[0, pl.ds(0, tail)], acc.at[didx_t],
                            add=True)
        plsc.subcore_barrier()
        pltpu.sync_copy(acc.at[pl.ds(rbase, rpt)],
                        out_ref.at[c, pl.ds(rbase, rpt)])

    return k(g, src, dst)


def _dinv_block(d0, d1):
    deg = d0 + d1
    return lax.rsqrt(jnp.maximum(deg, 1.0))


def _tc1_body(x_ref, w_ref, d0_ref, d1_ref, g_ref):
    dinv = _dinv_block(d0_ref[...], d1_ref[...])
    h = jnp.dot(x_ref[...], w_ref[...], preferred_element_type=jnp.float32)
    g_ref[...] = h * dinv


def _tc2_body(a0_ref, a1_ref, g1_ref, d0_ref, d1_ref, w_ref, b_ref, g2_ref):
    dinv = _dinv_block(d0_ref[...], d1_ref[...])
    x2 = jnp.maximum(
        dinv * (a0_ref[...] + a1_ref[...] + g1_ref[...]) + b_ref[...], 0.0)
    h = jnp.dot(x2, w_ref[...], preferred_element_type=jnp.float32)
    g2_ref[...] = h * dinv


def _tc3_body(a0_ref, a1_ref, g2_ref, d0_ref, d1_ref, b_ref, batch_ref,
              wc_ref, bc_ref, out_ref, sums, cnts):
    i = pl.program_id(0)
    dinv = _dinv_block(d0_ref[...], d1_ref[...])
    h3 = jnp.maximum(
        dinv * (a0_ref[...] + a1_ref[...] + g2_ref[...]) + b_ref[...], 0.0)
    bb = batch_ref[0, 0, :]
    onehot = jnp.equal(
        jnp.reshape(bb, (R, 1)),
        lax.broadcasted_iota(jnp.int32, (R, NG), 1)).astype(jnp.float32)
    ps = lax.dot_general(onehot, h3, (((0,), (0,)), ((), ())),
                         preferred_element_type=jnp.float32)
    pc = lax.dot_general(onehot, jnp.ones((R, D), jnp.float32),
                         (((0,), (0,)), ((), ())),
                         preferred_element_type=jnp.float32)

    @pl.when(i == 0)
    def _():
        sums[...] = ps
        cnts[...] = pc

    @pl.when(i > 0)
    def _():
        sums[...] += ps
        cnts[...] += pc

    @pl.when(i == GRID - 1)
    def _():
        pooled = sums[...] / jnp.maximum(cnts[...], 1.0)
        logits = jnp.dot(pooled, wc_ref[...],
                         preferred_element_type=jnp.float32) + bc_ref[...]
        m = jnp.max(logits, axis=1, keepdims=True)
        sh = logits - m
        lse = jnp.log(jnp.sum(jnp.exp(sh), axis=1, keepdims=True))
        out_ref[...] = sh - lse


def kernel(x, edge_index, batch, W1, b1, W2, b2, Wc, bc):
    src = edge_index[0]
    dst = edge_index[1]

    deg = _sc_degree(dst)
    d0 = jnp.reshape(deg[0], (DEG_PAD, 1))
    d1 = jnp.reshape(deg[1], (DEG_PAD, 1))

    row_spec = pl.BlockSpec((R, D), lambda i: (i, 0))
    dspec = pl.BlockSpec((R, 1), lambda i: (i, 0))
    wspec = pl.BlockSpec((D, D), lambda i: (0, 0))
    bspec = pl.BlockSpec((1, D), lambda i: (0, 0))

    g1 = pl.pallas_call(
        _tc1_body,
        grid=(GRID,),
        in_specs=[row_spec, wspec, dspec, dspec],
        out_specs=row_spec,
        out_shape=jax.ShapeDtypeStruct((N_NODES, D), jnp.float32),
    )(x, W1, d0, d1)

    a1 = _sc_aggregate(g1, src, dst)

    g2 = pl.pallas_call(
        _tc2_body,
        grid=(GRID,),
        in_specs=[row_spec, row_spec, row_spec, dspec, dspec, wspec, bspec],
        out_specs=row_spec,
        out_shape=jax.ShapeDtypeStruct((N_NODES, D), jnp.float32),
    )(a1[0], a1[1], g1, d0, d1, W2, jnp.reshape(b1, (1, D)))

    a2 = _sc_aggregate(g2, src, dst)

    batch3 = jnp.reshape(batch, (GRID, 1, R))
    wc_pad = jnp.zeros((D, D), jnp.float32).at[:, :Wc.shape[1]].set(Wc)
    bc_pad = jnp.full((1, D), -1e30, jnp.float32).at[0, :bc.shape[0]].set(bc)

    logits_pad = pl.pallas_call(
        _tc3_body,
        grid=(GRID,),
        in_specs=[row_spec, row_spec, row_spec, dspec, dspec, bspec,
                  pl.BlockSpec((1, 1, R), lambda i: (i, 0, 0)),
                  wspec, bspec],
        out_specs=pl.BlockSpec((NG, D), lambda i: (0, 0)),
        out_shape=jax.ShapeDtypeStruct((NG, D), jnp.float32),
        scratch_shapes=[pltpu.VMEM((NG, D), jnp.float32),
                        pltpu.VMEM((NG, D), jnp.float32)],
    )(a2[0], a2[1], g2, d0, d1, jnp.reshape(b2, (1, D)), batch3, wc_pad,
      bc_pad)

    return logits_pad[:, :bc.shape[0]]


# speedup vs baseline: 29.0540x; 1.1865x over previous
import functools

import jax
import jax.numpy as jnp
from jax import lax
from jax.experimental import pallas as pl
from jax.experimental.pallas import tpu as pltpu
from jax.experimental.pallas import tpu_sc as plsc

N_NODES = 10000
D = 128
NG = 64
NC = 2
NS = 16
NW = NC * NS
CH = 128

R = 400
GRID = N_NODES // R
N_PAD = 10240
DEG_PAD = 10240


def _sc_degree(dst2):
    nrows = dst2.shape[0]
    IS = 4
    mesh = plsc.VectorSubcoreMesh(core_axis_name="c", subcore_axis_name="s")

    @functools.partial(
        pl.kernel,
        out_type=jax.ShapeDtypeStruct((NC, DEG_PAD), jnp.float32),
        mesh=mesh,
        scratch_types=[
            pltpu.VMEM((640,), jnp.float32),
            pltpu.VMEM((CH,), jnp.float32),
            pltpu.VMEM((IS, CH), jnp.int32),
            pltpu.VMEM_SHARED((DEG_PAD,), jnp.float32),
            pltpu.SemaphoreType.DMA((IS,)),
            pltpu.SemaphoreType.DMA((IS,)),
        ],
    )
    def k(dst_ref, out_ref, zbuf, ones, didx, acc, jsem, ssem):
        c = lax.axis_index("c")
        s = lax.axis_index("s")
        wid = c * NS + s
        nch = (nrows - wid + NW - 1) // NW

        for p in range(2):
            pltpu.async_copy(dst_ref.at[wid + p * NW], didx.at[p],
                             jsem.at[p])

        def zfill(i, carry):
            zbuf[pl.ds(i * 16, 16)] = jnp.zeros((16,), jnp.float32)
            return carry

        lax.fori_loop(0, 40, zfill, 0)
        for i in range(CH // 16):
            ones[pl.ds(i * 16, 16)] = jnp.ones((16,), jnp.float32)
        pltpu.sync_copy(zbuf, acc.at[pl.ds(s * 640, 640)])
        plsc.subcore_barrier()

        def body(j, carry):
            b = j % IS
            m = wid + j * NW
            pltpu.make_async_copy(dst_ref.at[m], didx.at[b], jsem.at[b]).wait()
            pltpu.async_copy(ones, acc.at[didx.at[b]], ssem.at[b], add=True)

            @pl.when(j + 2 < nch)
            def _():
                bn = (j + 2) % IS

                @pl.when(j >= 2)
                def _():
                    bo = (j - 2) % IS
                    pltpu.make_async_copy(ones, acc.at[didx.at[bo]],
                                          ssem.at[bo]).wait()

                pltpu.async_copy(dst_ref.at[m + 2 * NW], didx.at[bn],
                                 jsem.at[bn])

            return carry

        lax.fori_loop(0, nch, body, 0)
        for dj in range(4):
            jj = nch - 4 + dj

            @pl.when(jj >= 0)
            def _():
                b = jj % IS
                pltpu.make_async_copy(ones, acc.at[didx.at[b]],
                                      ssem.at[b]).wait()

        plsc.subcore_barrier()
        pltpu.sync_copy(acc.at[pl.ds(s * 640, 640)],
                        out_ref.at[c, pl.ds(s * 640, 640)])

    return k(dst2)


def _sc_aggregate(g, src2, dst2):
    nrows = src2.shape[0]
    rpt = N_PAD // NS
    RS = 2
    IS = 4
    mesh = plsc.VectorSubcoreMesh(core_axis_name="c", subcore_axis_name="s")

    @functools.partial(
        pl.kernel,
        out_type=jax.ShapeDtypeStruct((NC, N_PAD, D), jnp.float32),
        mesh=mesh,
        scratch_types=[
            pltpu.VMEM((RS, CH, D), jnp.float32),
            pltpu.VMEM((IS, CH), jnp.int32),
            pltpu.VMEM((IS, CH), jnp.int32),
            pltpu.VMEM_SHARED((N_PAD, D), jnp.float32),
            pltpu.SemaphoreType.DMA((IS,)),
            pltpu.SemaphoreType.DMA((IS,)),
            pltpu.SemaphoreType.DMA((RS,)),
            pltpu.SemaphoreType.DMA((RS,)),
        ],
    )
    def k(g_ref, src_ref, dst_ref, out_ref, rows, sidx, didx, acc,
          isem, jsem, gsem, ssem):
        c = lax.axis_index("c")
        s = lax.axis_index("s")
        wid = c * NS + s
        nch = (nrows - wid + NW - 1) // NW

        for p in range(2):
            pltpu.async_copy(src_ref.at[wid + p * NW], sidx.at[p], isem.at[p])
            pltpu.async_copy(dst_ref.at[wid + p * NW], didx.at[p], jsem.at[p])

        def zrow(i, carry):
            for k16 in range(D // 16):
                rows[1, i, pl.ds(k16 * 16, 16)] = jnp.zeros((16,),
                                                            jnp.float32)
            return carry

        lax.fori_loop(0, CH, zrow, 0)
        rbase = s * rpt
        for t in range(rpt // CH):
            pltpu.sync_copy(rows.at[1], acc.at[pl.ds(rbase + t * CH, CH)])
        pltpu.make_async_copy(src_ref.at[wid], sidx.at[0], isem.at[0]).wait()
        pltpu.async_copy(g_ref.at[sidx.at[0]], rows.at[0], gsem.at[0])
        plsc.subcore_barrier()

        def body(j, carry):
            b = j % RS
            ib = j % IS

            @pl.when(j + 1 < nch)
            def _():
                bn = (j + 1) % RS
                ibn = (j + 1) % IS

                @pl.when(j >= 1)
                def _():
                    ibo = (j - 1) % IS
                    pltpu.make_async_copy(rows.at[bn], acc.at[didx.at[ibo]],
                                          ssem.at[bn]).wait()

                pltpu.make_async_copy(src_ref.at[wid + (j + 1) * NW],
                                      sidx.at[ibn], isem.at[ibn]).wait()
                pltpu.async_copy(g_ref.at[sidx.at[ibn]], rows.at[bn],
                                 gsem.at[bn])

            pltpu.make_async_copy(g_ref.at[sidx.at[ib]], rows.at[b],
                                  gsem.at[b]).wait()
            pltpu.make_async_copy(dst_ref.at[wid + j * NW], didx.at[ib],
                                  jsem.at[ib]).wait()
            pltpu.async_copy(rows.at[b], acc.at[didx.at[ib]], ssem.at[b],
                             add=True)

            @pl.when(j + 2 < nch)
            def _():
                ib2 = (j + 2) % IS
                pltpu.async_copy(src_ref.at[wid + (j + 2) * NW], sidx.at[ib2],
                                 isem.at[ib2])
                pltpu.async_copy(dst_ref.at[wid + (j + 2) * NW], didx.at[ib2],
                                 jsem.at[ib2])

            return carry

        lax.fori_loop(0, nch, body, 0)
        for dj in (nch - 2, nch - 1):
            b_d = dj % RS
            ib_d = dj % IS
            pltpu.make_async_copy(rows.at[b_d], acc.at[didx.at[ib_d]],
                                  ssem.at[b_d]).wait()
        plsc.subcore_barrier()
        pltpu.sync_copy(acc.at[pl.ds(rbase, rpt)],
                        out_ref.at[c, pl.ds(rbase, rpt)])

    return k(g, src2, dst2)


def _dinv_block(d0, d1):
    deg = d0 + d1
    return lax.rsqrt(jnp.maximum(deg, 1.0))


def _tc1_body(x_ref, w_ref, d0_ref, d1_ref, g_ref):
    dinv = _dinv_block(d0_ref[...], d1_ref[...])
    h = jnp.dot(x_ref[...], w_ref[...], preferred_element_type=jnp.float32)
    g_ref[...] = h * dinv


def _tc2_body(a0_ref, a1_ref, g1_ref, d0_ref, d1_ref, w_ref, b_ref, g2_ref):
    dinv = _dinv_block(d0_ref[...], d1_ref[...])
    x2 = jnp.maximum(
        dinv * (a0_ref[...] + a1_ref[...] + g1_ref[...]) + b_ref[...], 0.0)
    h = jnp.dot(x2, w_ref[...], preferred_element_type=jnp.float32)
    g2_ref[...] = h * dinv


def _tc3_body(a0_ref, a1_ref, g2_ref, d0_ref, d1_ref, b_ref, batch_ref,
              wc_ref, bc_ref, out_ref, sums, cnts):
    i = pl.program_id(0)
    dinv = _dinv_block(d0_ref[...], d1_ref[...])
    h3 = jnp.maximum(
        dinv * (a0_ref[...] + a1_ref[...] + g2_ref[...]) + b_ref[...], 0.0)
    bb = batch_ref[0, 0, :]
    onehot = jnp.equal(
        jnp.reshape(bb, (R, 1)),
        lax.broadcasted_iota(jnp.int32, (R, NG), 1)).astype(jnp.float32)
    ps = lax.dot_general(onehot, h3, (((0,), (0,)), ((), ())),
                         preferred_element_type=jnp.float32)
    pc = lax.dot_general(onehot, jnp.ones((R, D), jnp.float32),
                         (((0,), (0,)), ((), ())),
                         preferred_element_type=jnp.float32)

    @pl.when(i == 0)
    def _():
        sums[...] = ps
        cnts[...] = pc

    @pl.when(i > 0)
    def _():
        sums[...] += ps
        cnts[...] += pc

    @pl.when(i == GRID - 1)
    def _():
        pooled = sums[...] / jnp.maximum(cnts[...], 1.0)
        logits = jnp.dot(pooled, wc_ref[...],
                         preferred_element_type=jnp.float32) + bc_ref[...]
        m = jnp.max(logits, axis=1, keepdims=True)
        sh = logits - m
        lse = jnp.log(jnp.sum(jnp.exp(sh), axis=1, keepdims=True))
        out_ref[...] = sh - lse


def kernel(x, edge_index, batch, W1, b1, W2, b2, Wc, bc):
    E = edge_index.shape[1]
    src2 = jnp.reshape(edge_index[0], (E // CH, CH))
    dst2 = jnp.reshape(edge_index[1], (E // CH, CH))

    deg = _sc_degree(dst2)
    d0 = jnp.reshape(deg[0], (DEG_PAD, 1))
    d1 = jnp.reshape(deg[1], (DEG_PAD, 1))

    row_spec = pl.BlockSpec((R, D), lambda i: (i, 0))
    dspec = pl.BlockSpec((R, 1), lambda i: (i, 0))
    wspec = pl.BlockSpec((D, D), lambda i: (0, 0))
    bspec = pl.BlockSpec((1, D), lambda i: (0, 0))

    g1 = pl.pallas_call(
        _tc1_body,
        grid=(GRID,),
        in_specs=[row_spec, wspec, dspec, dspec],
        out_specs=row_spec,
        out_shape=jax.ShapeDtypeStruct((N_NODES, D), jnp.float32),
    )(x, W1, d0, d1)

    a1 = _sc_aggregate(g1, src2, dst2)

    g2 = pl.pallas_call(
        _tc2_body,
        grid=(GRID,),
        in_specs=[row_spec, row_spec, row_spec, dspec, dspec, wspec, bspec],
        out_specs=row_spec,
        out_shape=jax.ShapeDtypeStruct((N_NODES, D), jnp.float32),
    )(a1[0], a1[1], g1, d0, d1, W2, jnp.reshape(b1, (1, D)))

    a2 = _sc_aggregate(g2, src2, dst2)

    batch3 = jnp.reshape(batch, (GRID, 1, R))
    wc_pad = jnp.zeros((D, D), jnp.float32).at[:, :Wc.shape[1]].set(Wc)
    bc_pad = jnp.full((1, D), -1e30, jnp.float32).at[0, :bc.shape[0]].set(bc)

    logits_pad = pl.pallas_call(
        _tc3_body,
        grid=(GRID,),
        in_specs=[row_spec, row_spec, row_spec, dspec, dspec, bspec,
                  pl.BlockSpec((1, 1, R), lambda i: (i, 0, 0)),
                  wspec, bspec],
        out_specs=pl.BlockSpec((NG, D), lambda i: (0, 0)),
        out_shape=jax.ShapeDtypeStruct((NG, D), jnp.float32),
        scratch_shapes=[pltpu.VMEM((NG, D), jnp.float32),
                        pltpu.VMEM((NG, D), jnp.float32)],
    )(a2[0], a2[1], g2, d0, d1, jnp.reshape(b2, (1, D)), batch3, wc_pad,
      bc_pad)

    return logits_pad[:, :bc.shape[0]]


# speedup vs baseline: 30.3263x; 1.0438x over previous
import functools

import jax
import jax.numpy as jnp
from jax import lax
from jax.experimental import pallas as pl
from jax.experimental.pallas import tpu as pltpu
from jax.experimental.pallas import tpu_sc as plsc

N_NODES = 10000
D = 128
NG = 64
NC = 2
NS = 16
NW = NC * NS
CH = 128

R = 400
GRID = N_NODES // R
N_PAD = 10240
DEG_PAD = 10240


def _sc_degree(dst2):
    nrows = dst2.shape[0]
    IS = 4
    mesh = plsc.VectorSubcoreMesh(core_axis_name="c", subcore_axis_name="s")

    @functools.partial(
        pl.kernel,
        out_type=jax.ShapeDtypeStruct((NC, DEG_PAD), jnp.float32),
        mesh=mesh,
        scratch_types=[
            pltpu.VMEM((640,), jnp.float32),
            pltpu.VMEM((CH,), jnp.float32),
            pltpu.VMEM((IS, CH), jnp.int32),
            pltpu.VMEM_SHARED((DEG_PAD,), jnp.float32),
            pltpu.SemaphoreType.DMA((IS,)),
            pltpu.SemaphoreType.DMA((IS,)),
        ],
    )
    def k(dst_ref, out_ref, zbuf, ones, didx, acc, jsem, ssem):
        c = lax.axis_index("c")
        s = lax.axis_index("s")
        wid = c * NS + s
        nch = (nrows - wid + NW - 1) // NW

        for p in range(2):
            pltpu.async_copy(dst_ref.at[wid + p * NW], didx.at[p],
                             jsem.at[p])

        def zfill(i, carry):
            zbuf[pl.ds(i * 16, 16)] = jnp.zeros((16,), jnp.float32)
            return carry

        lax.fori_loop(0, 40, zfill, 0)
        for i in range(CH // 16):
            ones[pl.ds(i * 16, 16)] = jnp.ones((16,), jnp.float32)
        pltpu.sync_copy(zbuf, acc.at[pl.ds(s * 640, 640)])
        plsc.subcore_barrier()

        def body(j, carry):
            b = j % IS
            m = wid + j * NW
            pltpu.make_async_copy(dst_ref.at[m], didx.at[b], jsem.at[b]).wait()
            pltpu.async_copy(ones, acc.at[didx.at[b]], ssem.at[b], add=True)

            @pl.when(j + 2 < nch)
            def _():
                bn = (j + 2) % IS

                @pl.when(j >= 2)
                def _():
                    bo = (j - 2) % IS
                    pltpu.make_async_copy(ones, acc.at[didx.at[bo]],
                                          ssem.at[bo]).wait()

                pltpu.async_copy(dst_ref.at[m + 2 * NW], didx.at[bn],
                                 jsem.at[bn])

            return carry

        lax.fori_loop(0, nch, body, 0)
        for dj in range(4):
            jj = nch - 4 + dj

            @pl.when(jj >= 0)
            def _():
                b = jj % IS
                pltpu.make_async_copy(ones, acc.at[didx.at[b]],
                                      ssem.at[b]).wait()

        plsc.subcore_barrier()
        pltpu.sync_copy(acc.at[pl.ds(s * 640, 640)],
                        out_ref.at[c, pl.ds(s * 640, 640)])

    return k(dst2)


def _sc_aggregate(g, src2, dst2):
    nrows = src2.shape[0]
    rpt = N_PAD // NS
    RS = 2
    IS = 6
    mesh = plsc.VectorSubcoreMesh(core_axis_name="c", subcore_axis_name="s")

    @functools.partial(
        pl.kernel,
        out_type=jax.ShapeDtypeStruct((NC, N_PAD, D), jnp.float32),
        mesh=mesh,
        scratch_types=[
            pltpu.VMEM((RS, CH, D), jnp.float32),
            pltpu.VMEM((IS, CH), jnp.int32),
            pltpu.VMEM((IS, CH), jnp.int32),
            pltpu.VMEM_SHARED((N_PAD, D), jnp.float32),
            pltpu.SemaphoreType.DMA((IS,)),
            pltpu.SemaphoreType.DMA((IS,)),
            pltpu.SemaphoreType.DMA((RS,)),
            pltpu.SemaphoreType.DMA((RS,)),
        ],
    )
    def k(g_ref, src_ref, dst_ref, out_ref, rows, sidx, didx, acc,
          isem, jsem, gsem, ssem):
        c = lax.axis_index("c")
        s = lax.axis_index("s")
        wid = c * NS + s
        nch = (nrows - wid + NW - 1) // NW

        for p in range(2):
            pltpu.async_copy(src_ref.at[wid + p * NW], sidx.at[p], isem.at[p])
            pltpu.async_copy(dst_ref.at[wid + p * NW], didx.at[p], jsem.at[p])

        def zrow(i, carry):
            for k16 in range(D // 16):
                rows[RS - 1, i, pl.ds(k16 * 16, 16)] = jnp.zeros(
                    (16,), jnp.float32)
            return carry

        lax.fori_loop(0, CH, zrow, 0)
        rbase = s * rpt
        for t in range(rpt // CH):
            pltpu.sync_copy(rows.at[RS - 1], acc.at[pl.ds(rbase + t * CH, CH)])
        pltpu.make_async_copy(src_ref.at[wid], sidx.at[0], isem.at[0]).wait()
        pltpu.async_copy(g_ref.at[sidx.at[0]], rows.at[0], gsem.at[0])
        plsc.subcore_barrier()

        def body(j, carry):
            b = j % RS
            ib = j % IS

            @pl.when(j + 1 < nch)
            def _():
                bn = (j + 1) % RS
                ibn = (j + 1) % IS

                @pl.when(j + 1 - RS >= 0)
                def _():
                    jo = j + 1 - RS
                    pltpu.make_async_copy(rows.at[bn],
                                          acc.at[didx.at[jo % IS]],
                                          ssem.at[bn]).wait()

                pltpu.make_async_copy(src_ref.at[wid + (j + 1) * NW],
                                      sidx.at[ibn], isem.at[ibn]).wait()
                pltpu.async_copy(g_ref.at[sidx.at[ibn]], rows.at[bn],
                                 gsem.at[bn])

            pltpu.make_async_copy(g_ref.at[sidx.at[ib]], rows.at[b],
                                  gsem.at[b]).wait()
            pltpu.make_async_copy(dst_ref.at[wid + j * NW], didx.at[ib],
                                  jsem.at[ib]).wait()
            pltpu.async_copy(rows.at[b], acc.at[didx.at[ib]], ssem.at[b],
                             add=True)

            @pl.when(j + 2 < nch)
            def _():
                ib2 = (j + 2) % IS
                pltpu.async_copy(src_ref.at[wid + (j + 2) * NW], sidx.at[ib2],
                                 isem.at[ib2])
                pltpu.async_copy(dst_ref.at[wid + (j + 2) * NW], didx.at[ib2],
                                 jsem.at[ib2])

            return carry

        lax.fori_loop(0, nch, body, 0)
        for dj in range(RS):
            jj = nch - RS + dj

            @pl.when(jj >= 0)
            def _():
                pltpu.make_async_copy(rows.at[jj % RS],
                                      acc.at[didx.at[jj % IS]],
                                      ssem.at[jj % RS]).wait()

        plsc.subcore_barrier()
        pltpu.sync_copy(acc.at[pl.ds(rbase, rpt)],
                        out_ref.at[c, pl.ds(rbase, rpt)])

    return k(g, src2, dst2)


def _dinv_block(d0, d1):
    deg = d0 + d1
    return lax.rsqrt(jnp.maximum(deg, 1.0))


def _tc1_body(x_ref, w_ref, d0_ref, d1_ref, g_ref):
    dinv = _dinv_block(d0_ref[...], d1_ref[...])
    h = jnp.dot(x_ref[...], w_ref[...], preferred_element_type=jnp.float32)
    g_ref[...] = h * dinv


def _tc2_body(a_ref0, a_ref1, g1_ref, d0_ref, d1_ref, w_ref, b_ref, g2_ref):
    dinv = _dinv_block(d0_ref[...], d1_ref[...])
    agg = (a_ref0[0] + a_ref1[0] + g1_ref[...]).astype(jnp.float32)
    x2 = jnp.maximum(dinv * agg + b_ref[...], 0.0)
    h = jnp.dot(x2, w_ref[...], preferred_element_type=jnp.float32)
    g2_ref[...] = h * dinv


def _tc3_body(a_ref0, a_ref1, g2_ref, d0_ref, d1_ref, b_ref, batch_ref,
              wc_ref, bc_ref, out_ref, sums, cnts):
    i = pl.program_id(0)
    dinv = _dinv_block(d0_ref[...], d1_ref[...])
    agg = (a_ref0[0] + a_ref1[0] + g2_ref[...]).astype(jnp.float32)
    h3 = jnp.maximum(dinv * agg + b_ref[...], 0.0)
    bb = batch_ref[0, 0, :]
    onehot = jnp.equal(
        jnp.reshape(bb, (R, 1)),
        lax.broadcasted_iota(jnp.int32, (R, NG), 1)).astype(jnp.float32)
    ps = lax.dot_general(onehot, h3, (((0,), (0,)), ((), ())),
                         preferred_element_type=jnp.float32)
    pc = lax.dot_general(onehot, jnp.ones((R, D), jnp.float32),
                         (((0,), (0,)), ((), ())),
                         preferred_element_type=jnp.float32)

    @pl.when(i == 0)
    def _():
        sums[...] = ps
        cnts[...] = pc

    @pl.when(i > 0)
    def _():
        sums[...] += ps
        cnts[...] += pc

    @pl.when(i == GRID - 1)
    def _():
        pooled = sums[...] / jnp.maximum(cnts[...], 1.0)
        logits = jnp.dot(pooled, wc_ref[...],
                         preferred_element_type=jnp.float32) + bc_ref[...]
        m = jnp.max(logits, axis=1, keepdims=True)
        sh = logits - m
        lse = jnp.log(jnp.sum(jnp.exp(sh), axis=1, keepdims=True))
        out_ref[...] = sh - lse


def kernel(x, edge_index, batch, W1, b1, W2, b2, Wc, bc):
    E = edge_index.shape[1]
    src2 = jnp.reshape(edge_index[0], (E // CH, CH))
    dst2 = jnp.reshape(edge_index[1], (E // CH, CH))

    deg = _sc_degree(dst2)
    d0 = jnp.reshape(deg[0], (DEG_PAD, 1))
    d1 = jnp.reshape(deg[1], (DEG_PAD, 1))

    row_spec = pl.BlockSpec((R, D), lambda i: (i, 0))
    aspec0 = pl.BlockSpec((1, R, D), lambda i: (0, i, 0))
    aspec1 = pl.BlockSpec((1, R, D), lambda i: (1, i, 0))
    dspec = pl.BlockSpec((R, 1), lambda i: (i, 0))
    wspec = pl.BlockSpec((D, D), lambda i: (0, 0))
    bspec = pl.BlockSpec((1, D), lambda i: (0, 0))

    g1 = pl.pallas_call(
        _tc1_body,
        grid=(GRID,),
        in_specs=[row_spec, wspec, dspec, dspec],
        out_specs=row_spec,
        out_shape=jax.ShapeDtypeStruct((N_NODES, D), jnp.float32),
    )(x, W1, d0, d1)

    a1 = _sc_aggregate(g1, src2, dst2)

    g2 = pl.pallas_call(
        _tc2_body,
        grid=(GRID,),
        in_specs=[aspec0, aspec1, row_spec, dspec, dspec, wspec, bspec],
        out_specs=row_spec,
        out_shape=jax.ShapeDtypeStruct((N_NODES, D), jnp.float32),
    )(a1, a1, g1, d0, d1, W2, jnp.reshape(b1, (1, D)))

    a2 = _sc_aggregate(g2, src2, dst2)

    batch3 = jnp.reshape(batch, (GRID, 1, R))
    wc_pad = jnp.zeros((D, D), jnp.float32).at[:, :Wc.shape[1]].set(Wc)
    bc_pad = jnp.full((1, D), -1e30, jnp.float32).at[0, :bc.shape[0]].set(bc)

    logits_pad = pl.pallas_call(
        _tc3_body,
        grid=(GRID,),
        in_specs=[aspec0, aspec1, row_spec, dspec, dspec, bspec,
                  pl.BlockSpec((1, 1, R), lambda i: (i, 0, 0)),
                  wspec, bspec],
        out_specs=pl.BlockSpec((NG, D), lambda i: (0, 0)),
        out_shape=jax.ShapeDtypeStruct((NG, D), jnp.float32),
        scratch_shapes=[pltpu.VMEM((NG, D), jnp.float32),
                        pltpu.VMEM((NG, D), jnp.float32)],
    )(a2, a2, g2, d0, d1, jnp.reshape(b2, (1, D)), batch3, wc_pad, bc_pad)

    return logits_pad[:, :bc.shape[0]]


# speedup vs baseline: 34.0990x; 1.1244x over previous
import functools

import jax
import jax.numpy as jnp
from jax import lax
from jax.experimental import pallas as pl
from jax.experimental.pallas import tpu as pltpu
from jax.experimental.pallas import tpu_sc as plsc

N_NODES = 10000
D = 128
NG = 64
NC = 2
NS = 16
NW = NC * NS
CH = 128

R = 1000
GRID = N_NODES // R
N_PAD = 10240
DEG_PAD = 10240


def _sc_degree(edge_flat, E):
    nrows = E // CH
    IS = 4
    mesh = plsc.VectorSubcoreMesh(core_axis_name="c", subcore_axis_name="s")

    @functools.partial(
        pl.kernel,
        out_type=jax.ShapeDtypeStruct((NC, DEG_PAD), jnp.float32),
        mesh=mesh,
        scratch_types=[
            pltpu.VMEM((640,), jnp.float32),
            pltpu.VMEM((CH,), jnp.float32),
            pltpu.VMEM((IS, CH), jnp.int32),
            pltpu.VMEM_SHARED((DEG_PAD,), jnp.float32),
            pltpu.SemaphoreType.DMA((IS,)),
            pltpu.SemaphoreType.DMA((IS,)),
        ],
    )
    def k(dst_ref, out_ref, zbuf, ones, didx, acc, jsem, ssem):
        c = lax.axis_index("c")
        s = lax.axis_index("s")
        wid = c * NS + s
        nch = (nrows - wid + NW - 1) // NW

        for p in range(2):
            dbase = pl.multiple_of(E + (wid + p * NW) * CH, 8)
            pltpu.async_copy(dst_ref.at[pl.ds(dbase, CH)], didx.at[p],
                             jsem.at[p])

        def zfill(i, carry):
            zbuf[pl.ds(i * 16, 16)] = jnp.zeros((16,), jnp.float32)
            return carry

        lax.fori_loop(0, 40, zfill, 0)
        for i in range(CH // 16):
            ones[pl.ds(i * 16, 16)] = jnp.ones((16,), jnp.float32)
        pltpu.sync_copy(zbuf, acc.at[pl.ds(s * 640, 640)])
        plsc.subcore_barrier()

        def body(j, carry):
            b = j % IS
            mb = pl.multiple_of(E + (wid + j * NW) * CH, 8)
            pltpu.make_async_copy(dst_ref.at[pl.ds(mb, CH)], didx.at[b],
                                  jsem.at[b]).wait()
            pltpu.async_copy(ones, acc.at[didx.at[b]], ssem.at[b], add=True)

            @pl.when(j + 2 < nch)
            def _():
                bn = (j + 2) % IS

                @pl.when(j >= 2)
                def _():
                    bo = (j - 2) % IS
                    pltpu.make_async_copy(ones, acc.at[didx.at[bo]],
                                          ssem.at[bo]).wait()

                mb2 = pl.multiple_of(E + (wid + (j + 2) * NW) * CH, 8)
                pltpu.async_copy(dst_ref.at[pl.ds(mb2, CH)], didx.at[bn],
                                 jsem.at[bn])

            return carry

        lax.fori_loop(0, nch, body, 0)
        for dj in range(4):
            jj = nch - 4 + dj

            @pl.when(jj >= 0)
            def _():
                b = jj % IS
                pltpu.make_async_copy(ones, acc.at[didx.at[b]],
                                      ssem.at[b]).wait()

        plsc.subcore_barrier()
        pltpu.sync_copy(acc.at[pl.ds(s * 640, 640)],
                        out_ref.at[c, pl.ds(s * 640, 640)])

    return k(edge_flat)


def _sc_aggregate(g, edge_flat, E):
    nrows = E // CH
    rpt = N_PAD // NS
    RS = 2
    IS = 6
    mesh = plsc.VectorSubcoreMesh(core_axis_name="c", subcore_axis_name="s")

    @functools.partial(
        pl.kernel,
        out_type=jax.ShapeDtypeStruct((NC, N_PAD, D), jnp.float32),
        mesh=mesh,
        scratch_types=[
            pltpu.VMEM((RS, CH, D), jnp.float32),
            pltpu.VMEM((IS, CH), jnp.int32),
            pltpu.VMEM((IS, CH), jnp.int32),
            pltpu.VMEM_SHARED((N_PAD, D), jnp.float32),
            pltpu.SemaphoreType.DMA((IS,)),
            pltpu.SemaphoreType.DMA((IS,)),
            pltpu.SemaphoreType.DMA((RS,)),
            pltpu.SemaphoreType.DMA((RS,)),
        ],
    )
    def k(g_ref, e_ref, out_ref, rows, sidx, didx, acc,
          isem, jsem, gsem, ssem):
        c = lax.axis_index("c")
        s = lax.axis_index("s")
        wid = c * NS + s
        nch = (nrows - wid + NW - 1) // NW

        for p in range(2):
            sb = pl.multiple_of((wid + p * NW) * CH, 8)
            db = pl.multiple_of(E + (wid + p * NW) * CH, 8)
            pltpu.async_copy(e_ref.at[pl.ds(sb, CH)], sidx.at[p], isem.at[p])
            pltpu.async_copy(e_ref.at[pl.ds(db, CH)], didx.at[p], jsem.at[p])

        def zrow(i, carry):
            for k16 in range(D // 16):
                rows[RS - 1, i, pl.ds(k16 * 16, 16)] = jnp.zeros(
                    (16,), jnp.float32)
            return carry

        lax.fori_loop(0, CH, zrow, 0)
        rbase = s * rpt
        for t in range(rpt // CH):
            pltpu.sync_copy(rows.at[RS - 1], acc.at[pl.ds(rbase + t * CH, CH)])
        sb0 = pl.multiple_of(wid * CH, 8)
        pltpu.make_async_copy(e_ref.at[pl.ds(sb0, CH)], sidx.at[0],
                              isem.at[0]).wait()
        pltpu.async_copy(g_ref.at[sidx.at[0]], rows.at[0], gsem.at[0])
        plsc.subcore_barrier()

        def body(j, carry):
            b = j % RS
            ib = j % IS

            @pl.when(j + 1 < nch)
            def _():
                bn = (j + 1) % RS
                ibn = (j + 1) % IS

                @pl.when(j + 1 - RS >= 0)
                def _():
                    jo = j + 1 - RS
                    pltpu.make_async_copy(rows.at[bn],
                                          acc.at[didx.at[jo % IS]],
                                          ssem.at[bn]).wait()

                sb1 = pl.multiple_of((wid + (j + 1) * NW) * CH, 8)
                pltpu.make_async_copy(e_ref.at[pl.ds(sb1, CH)],
                                      sidx.at[ibn], isem.at[ibn]).wait()
                pltpu.async_copy(g_ref.at[sidx.at[ibn]], rows.at[bn],
                                 gsem.at[bn])

            pltpu.make_async_copy(g_ref.at[sidx.at[ib]], rows.at[b],
                                  gsem.at[b]).wait()
            db0 = pl.multiple_of(E + (wid + j * NW) * CH, 8)
            pltpu.make_async_copy(e_ref.at[pl.ds(db0, CH)], didx.at[ib],
                                  jsem.at[ib]).wait()
            pltpu.async_copy(rows.at[b], acc.at[didx.at[ib]], ssem.at[b],
                             add=True)

            @pl.when(j + 2 < nch)
            def _():
                ib2 = (j + 2) % IS
                sb2 = pl.multiple_of((wid + (j + 2) * NW) * CH, 8)
                db2 = pl.multiple_of(E + (wid + (j + 2) * NW) * CH, 8)
                pltpu.async_copy(e_ref.at[pl.ds(sb2, CH)], sidx.at[ib2],
                                 isem.at[ib2])
                pltpu.async_copy(e_ref.at[pl.ds(db2, CH)], didx.at[ib2],
                                 jsem.at[ib2])

            return carry

        lax.fori_loop(0, nch, body, 0)
        for dj in range(RS):
            jj = nch - RS + dj

            @pl.when(jj >= 0)
            def _():
                pltpu.make_async_copy(rows.at[jj % RS],
                                      acc.at[didx.at[jj % IS]],
                                      ssem.at[jj % RS]).wait()

        plsc.subcore_barrier()
        pltpu.sync_copy(acc.at[pl.ds(rbase, rpt)],
                        out_ref.at[c, pl.ds(rbase, rpt)])

    return k(g, edge_flat)


def _dinv_block(d0, d1):
    deg = d0 + d1
    return lax.rsqrt(jnp.maximum(deg, 1.0))


def _tc1_body(x_ref, w_ref, d0_ref, d1_ref, g_ref):
    dinv = _dinv_block(d0_ref[...], d1_ref[...])
    h = jnp.dot(x_ref[...].astype(jnp.bfloat16), w_ref[...],
                preferred_element_type=jnp.float32)
    g_ref[...] = h * dinv


def _tc2_body(a_ref0, a_ref1, g1_ref, d0_ref, d1_ref, w_ref, b_ref, g2_ref):
    dinv = _dinv_block(d0_ref[...], d1_ref[...])
    agg = (a_ref0[0] + a_ref1[0] + g1_ref[...]).astype(jnp.float32)
    x2 = jnp.maximum(dinv * agg + b_ref[...], 0.0)
    h = jnp.dot(x2.astype(jnp.bfloat16), w_ref[...],
                preferred_element_type=jnp.float32)
    g2_ref[...] = h * dinv


def _tc3_body(a_ref0, a_ref1, g2_ref, d0_ref, d1_ref, b_ref, batch_ref,
              wc_ref, bc_ref, out_ref, sums, cnts):
    i = pl.program_id(0)
    dinv = _dinv_block(d0_ref[...], d1_ref[...])
    agg = (a_ref0[0] + a_ref1[0] + g2_ref[...]).astype(jnp.float32)
    h3 = jnp.maximum(dinv * agg + b_ref[...], 0.0)
    bb = batch_ref[0, 0, :]
    onehot = jnp.equal(
        jnp.reshape(bb, (R, 1)),
        lax.broadcasted_iota(jnp.int32, (R, NG), 1)).astype(jnp.float32)
    ps = lax.dot_general(onehot, h3, (((0,), (0,)), ((), ())),
                         preferred_element_type=jnp.float32)
    pc = lax.dot_general(onehot, jnp.ones((R, D), jnp.float32),
                         (((0,), (0,)), ((), ())),
                         preferred_element_type=jnp.float32)

    @pl.when(i == 0)
    def _():
        sums[...] = ps
        cnts[...] = pc

    @pl.when(i > 0)
    def _():
        sums[...] += ps
        cnts[...] += pc

    @pl.when(i == GRID - 1)
    def _():
        pooled = sums[...] / jnp.maximum(cnts[...], 1.0)
        logits = jnp.dot(pooled, wc_ref[...],
                         preferred_element_type=jnp.float32) + bc_ref[...]
        m = jnp.max(logits, axis=1, keepdims=True)
        sh = logits - m
        lse = jnp.log(jnp.sum(jnp.exp(sh), axis=1, keepdims=True))
        out_ref[...] = sh - lse


def kernel(x, edge_index, batch, W1, b1, W2, b2, Wc, bc):
    E = edge_index.shape[1]
    edge_flat = jnp.reshape(edge_index, (2 * E,))

    deg = _sc_degree(edge_flat, E)
    d0 = jnp.reshape(deg[0], (DEG_PAD, 1))
    d1 = jnp.reshape(deg[1], (DEG_PAD, 1))

    row_spec = pl.BlockSpec((R, D), lambda i: (i, 0))
    aspec0 = pl.BlockSpec((1, R, D), lambda i: (0, i, 0))
    aspec1 = pl.BlockSpec((1, R, D), lambda i: (1, i, 0))
    dspec = pl.BlockSpec((R, 1), lambda i: (i, 0))
    wspec = pl.BlockSpec((D, D), lambda i: (0, 0))
    bspec = pl.BlockSpec((1, D), lambda i: (0, 0))

    g1 = pl.pallas_call(
        _tc1_body,
        grid=(GRID,),
        in_specs=[row_spec, wspec, dspec, dspec],
        out_specs=row_spec,
        out_shape=jax.ShapeDtypeStruct((N_NODES, D), jnp.float32),
    )(x, W1.astype(jnp.bfloat16), d0, d1)

    a1 = _sc_aggregate(g1, edge_flat, E)

    g2 = pl.pallas_call(
        _tc2_body,
        grid=(GRID,),
        in_specs=[aspec0, aspec1, row_spec, dspec, dspec, wspec, bspec],
        out_specs=row_spec,
        out_shape=jax.ShapeDtypeStruct((N_NODES, D), jnp.float32),
    )(a1, a1, g1, d0, d1, W2.astype(jnp.bfloat16), jnp.reshape(b1, (1, D)))

    a2 = _sc_aggregate(g2, edge_flat, E)

    batch3 = jnp.reshape(batch, (GRID, 1, R))
    wc_pad = jnp.zeros((D, D), jnp.float32).at[:, :Wc.shape[1]].set(Wc)
    bc_pad = jnp.full((1, D), -1e30, jnp.float32).at[0, :bc.shape[0]].set(bc)

    logits_pad = pl.pallas_call(
        _tc3_body,
        grid=(GRID,),
        in_specs=[aspec0, aspec1, row_spec, dspec, dspec, bspec,
                  pl.BlockSpec((1, 1, R), lambda i: (i, 0, 0)),
                  wspec, bspec],
        out_specs=pl.BlockSpec((NG, D), lambda i: (0, 0)),
        out_shape=jax.ShapeDtypeStruct((NG, D), jnp.float32),
        scratch_shapes=[pltpu.VMEM((NG, D), jnp.float32),
                        pltpu.VMEM((NG, D), jnp.float32)],
    )(a2, a2, g2, d0, d1, jnp.reshape(b2, (1, D)), batch3, wc_pad, bc_pad)

    return logits_pad[:, :bc.shape[0]]
